# Initial kernel scaffold; baseline (speedup 1.0000x reference)
import jax
import jax.numpy as jnp
from jax.experimental import pallas as pl


def kernel(user_feat, item_feat, edge_index, W2, b2, W3, b3, W4, b4, W5, b5, Wp, bp):
    raise NotImplementedError("write your pallas kernel here")


# speedup vs baseline: 3.0565x; 3.0565x over previous
import jax
import jax.numpy as jnp
import numpy as np
from jax import lax
from jax.experimental import pallas as pl
from jax.experimental.pallas import tpu as pltpu
from jax.experimental.pallas import tpu_sc as plsc

N_U = 5000
N_I = 5000
E = 160000
D = 256
NSUB = 16
CR = 5376
CC = 5120
RT = 24
NPASS = CR // (RT * 32)
CHK = 2000
NCHK = E // CHK
GPC = CHK // 16

_Z = np.int32(0)
_MESH = plsc.VectorSubcoreMesh(core_axis_name="c", subcore_axis_name="s")
_NLP = pltpu.CompilerParams(needs_layout_passes=False)


def _cbuild_body(dsts, srcs, z24, cout,
                 d0, s0, d1, s1, acc, sem0, sem1):
    c = lax.axis_index("c").astype(jnp.int32)
    s = lax.axis_index("s").astype(jnp.int32)
    w = s * jnp.int32(2) + c
    ones16 = jnp.full((16,), 1.0, jnp.float32)

    def load(g, db, sb, sm):
        off = pl.multiple_of(g * jnp.int32(CHK), CHK)
        pltpu.async_copy(dsts.at[pl.ds(off, CHK)], db, sm)
        pltpu.async_copy(srcs.at[pl.ds(off, CHK)], sb, sm)

    def drain(db, sb, sm):
        pltpu.make_async_copy(dsts.at[pl.ds(jnp.int32(0), CHK)], db, sm).wait()
        pltpu.make_async_copy(srcs.at[pl.ds(jnp.int32(0), CHK)], sb, sm).wait()

    def pass_body(p, carry):
        slot = p * jnp.int32(32) + w
        row0 = slot * jnp.int32(RT)
        pltpu.sync_copy(z24, acc)
        row0v = jnp.zeros((16,), jnp.int32) + row0

        def process(db, sb):
            def grp(j, cc2):
                sl = pl.ds(j * jnp.int32(16), 16)
                dv = db[sl]
                sv = sb[sl]
                lr = dv - row0v
                mask = (lr >= 0) & (lr < jnp.int32(RT))
                plsc.addupdate_scatter(acc, [lr, sv], ones16, mask=mask)
                return cc2

            lax.fori_loop(jnp.int32(0), jnp.int32(GPC), grp, jnp.int32(0))

        load(jnp.int32(0), d0, s0, sem0)

        def two(gg, cc2):
            g0 = gg * jnp.int32(2)
            load(g0 + jnp.int32(1), d1, s1, sem1)
            drain(d0, s0, sem0)
            process(d0, s0)

            @pl.when(gg < jnp.int32(NCHK // 2 - 1))
            def _():
                load(g0 + jnp.int32(2), d0, s0, sem0)

            drain(d1, s1, sem1)
            process(d1, s1)
            return cc2

        lax.fori_loop(jnp.int32(0), jnp.int32(NCHK // 2), two, jnp.int32(0))
        wout = pl.multiple_of(row0, 8)
        pltpu.sync_copy(acc, cout.at[pl.ds(wout, RT)])
        return carry

    lax.fori_loop(jnp.int32(0), jnp.int32(NPASS), pass_body, jnp.int32(0))


_cbuild = pl.kernel(
    _cbuild_body,
    out_type=jax.ShapeDtypeStruct((CR, CC), jnp.float32),
    mesh=_MESH,
    compiler_params=_NLP,
    scratch_types=[
        pltpu.VMEM((CHK,), jnp.int32),
        pltpu.VMEM((CHK,), jnp.int32),
        pltpu.VMEM((CHK,), jnp.int32),
        pltpu.VMEM((CHK,), jnp.int32),
        pltpu.VMEM((RT, CC), jnp.float32),
        pltpu.SemaphoreType.DMA,
        pltpu.SemaphoreType.DMA,
    ],
)


def _mm_n_kern(c_ref, x_ref, o_ref):
    o_ref[:] = jnp.dot(c_ref[:], x_ref[:], preferred_element_type=jnp.float32)


def _mm_t_kern(c_ref, x_ref, o_ref):
    o_ref[:] = lax.dot_general(c_ref[:], x_ref[:],
                               dimension_numbers=(((0,), (0,)), ((), ())),
                               preferred_element_type=jnp.float32)


_BMN = 384
_BMT = 512


def _make_mm_n(n):
    return pl.pallas_call(
        _mm_n_kern,
        grid=(CR // _BMN,),
        in_specs=[pl.BlockSpec((_BMN, CC), lambda i: (i, _Z)),
                  pl.BlockSpec((CC, n), lambda i: (_Z, _Z))],
        out_specs=pl.BlockSpec((_BMN, n), lambda i: (i, _Z)),
        out_shape=jax.ShapeDtypeStruct((CR, n), jnp.float32),
    )


def _make_mm_t(n):
    return pl.pallas_call(
        _mm_t_kern,
        grid=(CC // _BMT,),
        in_specs=[pl.BlockSpec((CR, _BMT), lambda i: (_Z, i)),
                  pl.BlockSpec((CR, n), lambda i: (_Z, _Z))],
        out_specs=pl.BlockSpec((_BMT, n), lambda i: (i, _Z)),
        out_shape=jax.ShapeDtypeStruct((CC, n), jnp.float32),
    )


_mm_n384 = _make_mm_n(384)
_mm_t384 = _make_mm_t(384)
_mm_n256 = _make_mm_n(256)
_mm_t256 = _make_mm_t(256)


def _layer_kern(x_ref, s_ref, c_ref, wa_ref, wb_ref, b_ref, o_ref):
    inv = 1.0 / jnp.maximum(c_ref[:, 0:1], 1.0)
    h = s_ref[:] * inv
    y = jnp.dot(x_ref[:], wa_ref[:], preferred_element_type=jnp.float32)
    y = y + jnp.dot(h, wb_ref[:], preferred_element_type=jnp.float32)
    o_ref[:] = jnp.maximum(y + b_ref[:], 0.0)


def _proj_kern(x_ref, s_ref, c_ref, wa_ref, wb_ref, b_ref, wp_ref, pb_ref,
               o_ref):
    inv = 1.0 / jnp.maximum(c_ref[:, 0:1], 1.0)
    h = s_ref[:] * inv
    y = jnp.dot(x_ref[:], wa_ref[:], preferred_element_type=jnp.float32)
    y = y + jnp.dot(h, wb_ref[:], preferred_element_type=jnp.float32)
    y = y + b_ref[:]
    o_ref[:] = jnp.dot(y, wp_ref[:], preferred_element_type=jnp.float32) + pb_ref[:]


_BN = 1000


def _node_specs():
    return [
        pl.BlockSpec((_BN, D), lambda i: (i, _Z)),
        pl.BlockSpec((_BN, D), lambda i: (i, _Z)),
        pl.BlockSpec((_BN, 16), lambda i: (i, _Z)),
        pl.BlockSpec((D, D), lambda i: (_Z, _Z)),
        pl.BlockSpec((D, D), lambda i: (_Z, _Z)),
        pl.BlockSpec((1, D), lambda i: (_Z, _Z)),
    ]


_layer = pl.pallas_call(
    _layer_kern,
    grid=(N_U // _BN,),
    in_specs=_node_specs(),
    out_specs=pl.BlockSpec((_BN, D), lambda i: (i, _Z)),
    out_shape=jax.ShapeDtypeStruct((N_U, D), jnp.float32),
)

_proj = pl.pallas_call(
    _proj_kern,
    grid=(N_U // _BN,),
    in_specs=_node_specs() + [
        pl.BlockSpec((D, 16), lambda i: (_Z, _Z)),
        pl.BlockSpec((1, 16), lambda i: (_Z, _Z)),
    ],
    out_specs=pl.BlockSpec((_BN, 16), lambda i: (i, _Z)),
    out_shape=jax.ShapeDtypeStruct((N_U, 16), jnp.float32),
)


KP = 256
NCHP = E // KP
NPT = 4 * CC


def _pred_body(ptab, srcs, dsts, out, tab_v, si_v, di_v, ob_v, sem):
    c = lax.axis_index("c").astype(jnp.int32)
    s = lax.axis_index("s").astype(jnp.int32)
    w = s * jnp.int32(2) + c
    pltpu.sync_copy(ptab, tab_v)

    def step(i, carry):
        q = i * jnp.int32(2 * NSUB) + w

        @pl.when(q < jnp.int32(NCHP))
        def _():
            _pred_chunk(q, tab_v, si_v, di_v, ob_v, srcs, dsts, out, sem)
        return carry

    lax.fori_loop(jnp.int32(0), jnp.int32((NCHP + 2 * NSUB - 1) // (2 * NSUB)),
                  step, jnp.int32(0))


def _pred_chunk(q, tab_v, si_v, di_v, ob_v, srcs, dsts, out, sem):
    off = pl.multiple_of(q * jnp.int32(KP), KP)
    d1 = pltpu.async_copy(srcs.at[pl.ds(off, KP)], si_v, sem)
    d2 = pltpu.async_copy(dsts.at[pl.ds(off, KP)], di_v, sem)
    d1.wait()
    d2.wait()

    def inner(j, carry2):
        sv = si_v[pl.ds(j * jnp.int32(16), 16)]
        dv = di_v[pl.ds(j * jnp.int32(16), 16)]
        l0 = (plsc.load_gather(tab_v, [sv])
              + plsc.load_gather(tab_v, [dv + jnp.int32(2 * CC)]))
        l1 = (plsc.load_gather(tab_v, [sv + jnp.int32(CC)])
              + plsc.load_gather(tab_v, [dv + jnp.int32(3 * CC)]))
        m = jnp.maximum(l0, l1)
        e0 = jnp.exp(l0 - m)
        e1 = jnp.exp(l1 - m)
        t = e0 + e1
        jj = j * jnp.int32(32) + lax.iota(jnp.int32, 16) * jnp.int32(2)
        plsc.store_scatter(ob_v, [jj], e0 / t)
        plsc.store_scatter(ob_v, [jj + jnp.int32(1)], e1 / t)
        return carry2

    lax.fori_loop(jnp.int32(0), jnp.int32(KP // 16), inner, jnp.int32(0))
    pltpu.sync_copy(ob_v, out.at[pl.ds(off * 2, 2 * KP)])


_pred = pl.kernel(
    _pred_body,
    out_type=jax.ShapeDtypeStruct((2 * E,), jnp.float32),
    mesh=_MESH,
    compiler_params=_NLP,
    scratch_types=[
        pltpu.VMEM((NPT,), jnp.float32),
        pltpu.VMEM((KP,), jnp.int32),
        pltpu.VMEM((KP,), jnp.int32),
        pltpu.VMEM((2 * KP,), jnp.float32),
        pltpu.SemaphoreType.DMA,
    ],
)


def _pad_rows(x, rows):
    return jnp.pad(x, ((0, rows - x.shape[0]), (0, 0)))


def kernel(user_feat, item_feat, edge_index, W2, b2, W3, b3, W4, b4, W5, b5,
           Wp, bp):
    src = edge_index[0].astype(jnp.int32)
    dst = edge_index[1].astype(jnp.int32)
    z24 = jnp.zeros((RT, CC), jnp.float32)

    cm = _cbuild(dst, src, z24)

    ones_u = jnp.ones((N_U, 128), jnp.float32)
    xu_aug = _pad_rows(jnp.concatenate([user_feat, ones_u], axis=1), CC)
    xi_aug = _pad_rows(jnp.concatenate([item_feat, ones_u], axis=1), CR)
    si = _mm_n384(cm, xu_aug)
    su = _mm_t384(cm, xi_aug)
    sums_i = si[:N_I, :D]
    cnt_i = si[:N_I, D:D + 16]
    sums_u = su[:N_U, :D]
    cnt_u = su[:N_U, D:D + 16]

    u1 = _layer(user_feat, sums_u, cnt_u, W2[:D], W2[D:], b2.reshape(1, D))
    i1 = _layer(item_feat, sums_i, cnt_i, W3[:D], W3[D:], b3.reshape(1, D))

    si2 = _mm_n256(cm, _pad_rows(u1, CC))
    su2 = _mm_t256(cm, _pad_rows(i1, CR))

    wpa = jnp.pad(Wp[:D], ((0, 0), (0, 14)))
    wpb = jnp.pad(Wp[D:], ((0, 0), (0, 14)))
    bpu = jnp.pad(bp, (0, 14)).reshape(1, 16)
    zb = jnp.zeros((1, 16), jnp.float32)
    pu = _proj(u1, su2[:N_U], cnt_u, W4[:D], W4[D:], b4.reshape(1, D), wpa, bpu)
    pi = _proj(i1, si2[:N_I], cnt_i, W5[:D], W5[D:], b5.reshape(1, D), wpb, zb)

    pad = (0, CC - N_U)
    ptab = jnp.concatenate([
        jnp.pad(pu[:, 0], pad), jnp.pad(pu[:, 1], pad),
        jnp.pad(pi[:, 0], pad), jnp.pad(pi[:, 1], pad)])
    flat = _pred(ptab, src, dst)
    return flat.reshape(E, 2)


# speedup vs baseline: 4.5924x; 1.5025x over previous
import jax
import jax.numpy as jnp
import numpy as np
from jax import lax
from jax.experimental import pallas as pl
from jax.experimental.pallas import tpu as pltpu
from jax.experimental.pallas import tpu_sc as plsc

N_U = 5000
N_I = 5000
E = 160000
D = 256
NSUB = 16
CR = 5376
CC = 5120
PW = CC // 4
RT = 96
NSLOT = CR // RT
NPASS = 2
CHK = 2000
NCHK = E // CHK
GPC = CHK // 16

_Z = np.int32(0)
_MESH = plsc.VectorSubcoreMesh(core_axis_name="c", subcore_axis_name="s")
_NLP = pltpu.CompilerParams(needs_layout_passes=False)


def _cbuild_body(dsts, srcs, z24, cout,
                 d0, s0, d1, s1, acc, sem0, sem1):
    c = lax.axis_index("c").astype(jnp.int32)
    s = lax.axis_index("s").astype(jnp.int32)
    w = s * jnp.int32(2) + c
    ones16 = jnp.full((16,), 1.0, jnp.float32)

    def load(g, db, sb, sm):
        off = pl.multiple_of(g * jnp.int32(CHK), CHK)
        pltpu.async_copy(dsts.at[pl.ds(off, CHK)], db, sm)
        pltpu.async_copy(srcs.at[pl.ds(off, CHK)], sb, sm)

    def drain(db, sb, sm):
        pltpu.make_async_copy(dsts.at[pl.ds(jnp.int32(0), CHK)], db, sm).wait()
        pltpu.make_async_copy(srcs.at[pl.ds(jnp.int32(0), CHK)], sb, sm).wait()

    def pass_body(p, carry):
        slot = p * jnp.int32(32) + w

        @pl.when(slot < jnp.int32(NSLOT))
        def _():
            row0 = slot * jnp.int32(RT)
            pltpu.sync_copy(z24, acc)
            row0v = jnp.zeros((16,), jnp.int32) + row0
            one_i = jnp.full((16,), 1, jnp.int32)

            def process(db, sb):
                def grp(j, cc2):
                    sl = pl.ds(j * jnp.int32(16), 16)
                    dv = db[sl]
                    sv = sb[sl]
                    lr = dv - row0v
                    mask = (lr >= 0) & (lr < jnp.int32(RT))
                    wcol = lax.shift_right_logical(sv, jnp.int32(2))
                    val = lax.shift_left(
                        one_i, (sv & jnp.int32(3)) * jnp.int32(8))
                    plsc.addupdate_scatter(acc, [lr, wcol], val, mask=mask)
                    return cc2

                lax.fori_loop(jnp.int32(0), jnp.int32(GPC), grp, jnp.int32(0))

            load(jnp.int32(0), d0, s0, sem0)

            def two(gg, cc2):
                g0 = gg * jnp.int32(2)
                load(g0 + jnp.int32(1), d1, s1, sem1)
                drain(d0, s0, sem0)
                process(d0, s0)

                @pl.when(gg < jnp.int32(NCHK // 2 - 1))
                def _():
                    load(g0 + jnp.int32(2), d0, s0, sem0)

                drain(d1, s1, sem1)
                process(d1, s1)
                return cc2

            lax.fori_loop(jnp.int32(0), jnp.int32(NCHK // 2), two, jnp.int32(0))
            wout = pl.multiple_of(slot * jnp.int32(RT), 8)
            pltpu.sync_copy(acc, cout.at[pl.ds(wout, RT)])
        return carry

    lax.fori_loop(jnp.int32(0), jnp.int32(NPASS), pass_body, jnp.int32(0))


_cbuild = pl.kernel(
    _cbuild_body,
    out_type=jax.ShapeDtypeStruct((CR, PW), jnp.int32),
    mesh=_MESH,
    compiler_params=_NLP,
    scratch_types=[
        pltpu.VMEM((CHK,), jnp.int32),
        pltpu.VMEM((CHK,), jnp.int32),
        pltpu.VMEM((CHK,), jnp.int32),
        pltpu.VMEM((CHK,), jnp.int32),
        pltpu.VMEM((RT, PW), jnp.int32),
        pltpu.SemaphoreType.DMA,
        pltpu.SemaphoreType.DMA,
    ],
)


def _mm_n_kern(c_ref, x_ref, o_ref):
    o_ref[:] = jnp.dot(c_ref[:], x_ref[:], preferred_element_type=jnp.float32)


def _mm_t_kern(c_ref, x_ref, o_ref):
    o_ref[:] = lax.dot_general(c_ref[:], x_ref[:],
                               dimension_numbers=(((0,), (0,)), ((), ())),
                               preferred_element_type=jnp.float32)


_BMN = 384
_BMT = 512


def _make_mm_n(n):
    return pl.pallas_call(
        _mm_n_kern,
        grid=(CR // _BMN,),
        in_specs=[pl.BlockSpec((_BMN, CC), lambda i: (i, _Z)),
                  pl.BlockSpec((CC, n), lambda i: (_Z, _Z))],
        out_specs=pl.BlockSpec((_BMN, n), lambda i: (i, _Z)),
        out_shape=jax.ShapeDtypeStruct((CR, n), jnp.float32),
    )


def _make_mm_t(n):
    return pl.pallas_call(
        _mm_t_kern,
        grid=(CC // _BMT,),
        in_specs=[pl.BlockSpec((CR, _BMT), lambda i: (_Z, i)),
                  pl.BlockSpec((CR, n), lambda i: (_Z, _Z))],
        out_specs=pl.BlockSpec((_BMT, n), lambda i: (i, _Z)),
        out_shape=jax.ShapeDtypeStruct((CC, n), jnp.float32),
    )


_mm_n384 = _make_mm_n(384)
_mm_t384 = _make_mm_t(384)
_mm_n256 = _make_mm_n(256)
_mm_t256 = _make_mm_t(256)


def _layer_kern(x_ref, s_ref, c_ref, wa_ref, wb_ref, b_ref, o_ref):
    inv = 1.0 / jnp.maximum(c_ref[:, 0:1], 1.0)
    h = s_ref[:] * inv
    y = jnp.dot(x_ref[:], wa_ref[:], preferred_element_type=jnp.float32)
    y = y + jnp.dot(h, wb_ref[:], preferred_element_type=jnp.float32)
    o_ref[:] = jnp.maximum(y + b_ref[:], 0.0)


def _proj_kern(x_ref, s_ref, c_ref, wa_ref, wb_ref, b_ref, wp_ref, pb_ref,
               o_ref):
    inv = 1.0 / jnp.maximum(c_ref[:, 0:1], 1.0)
    h = s_ref[:] * inv
    y = jnp.dot(x_ref[:], wa_ref[:], preferred_element_type=jnp.float32)
    y = y + jnp.dot(h, wb_ref[:], preferred_element_type=jnp.float32)
    y = y + b_ref[:]
    o_ref[:] = jnp.dot(y, wp_ref[:], preferred_element_type=jnp.float32) + pb_ref[:]


_BN = 1000


def _node_specs():
    return [
        pl.BlockSpec((_BN, D), lambda i: (i, _Z)),
        pl.BlockSpec((_BN, D), lambda i: (i, _Z)),
        pl.BlockSpec((_BN, 16), lambda i: (i, _Z)),
        pl.BlockSpec((D, D), lambda i: (_Z, _Z)),
        pl.BlockSpec((D, D), lambda i: (_Z, _Z)),
        pl.BlockSpec((1, D), lambda i: (_Z, _Z)),
    ]


_layer = pl.pallas_call(
    _layer_kern,
    grid=(N_U // _BN,),
    in_specs=_node_specs(),
    out_specs=pl.BlockSpec((_BN, D), lambda i: (i, _Z)),
    out_shape=jax.ShapeDtypeStruct((N_U, D), jnp.float32),
)

_proj = pl.pallas_call(
    _proj_kern,
    grid=(N_U // _BN,),
    in_specs=_node_specs() + [
        pl.BlockSpec((D, 16), lambda i: (_Z, _Z)),
        pl.BlockSpec((1, 16), lambda i: (_Z, _Z)),
    ],
    out_specs=pl.BlockSpec((_BN, 16), lambda i: (i, _Z)),
    out_shape=jax.ShapeDtypeStruct((N_U, 16), jnp.float32),
)


KP = 256
NCHP = E // KP
NPT = 4 * CC


def _pred_body(ptab, srcs, dsts, out, tab_v, si_v, di_v, ob_v, sem):
    c = lax.axis_index("c").astype(jnp.int32)
    s = lax.axis_index("s").astype(jnp.int32)
    w = s * jnp.int32(2) + c
    pltpu.sync_copy(ptab, tab_v)

    def step(i, carry):
        q = i * jnp.int32(2 * NSUB) + w

        @pl.when(q < jnp.int32(NCHP))
        def _():
            _pred_chunk(q, tab_v, si_v, di_v, ob_v, srcs, dsts, out, sem)
        return carry

    lax.fori_loop(jnp.int32(0), jnp.int32((NCHP + 2 * NSUB - 1) // (2 * NSUB)),
                  step, jnp.int32(0))


def _pred_chunk(q, tab_v, si_v, di_v, ob_v, srcs, dsts, out, sem):
    off = pl.multiple_of(q * jnp.int32(KP), KP)
    d1 = pltpu.async_copy(srcs.at[pl.ds(off, KP)], si_v, sem)
    d2 = pltpu.async_copy(dsts.at[pl.ds(off, KP)], di_v, sem)
    d1.wait()
    d2.wait()

    def inner(j, carry2):
        sv = si_v[pl.ds(j * jnp.int32(16), 16)]
        dv = di_v[pl.ds(j * jnp.int32(16), 16)]
        l0 = (plsc.load_gather(tab_v, [sv])
              + plsc.load_gather(tab_v, [dv + jnp.int32(2 * CC)]))
        l1 = (plsc.load_gather(tab_v, [sv + jnp.int32(CC)])
              + plsc.load_gather(tab_v, [dv + jnp.int32(3 * CC)]))
        m = jnp.maximum(l0, l1)
        e0 = jnp.exp(l0 - m)
        e1 = jnp.exp(l1 - m)
        t = e0 + e1
        jj = j * jnp.int32(32) + lax.iota(jnp.int32, 16) * jnp.int32(2)
        plsc.store_scatter(ob_v, [jj], e0 / t)
        plsc.store_scatter(ob_v, [jj + jnp.int32(1)], e1 / t)
        return carry2

    lax.fori_loop(jnp.int32(0), jnp.int32(KP // 16), inner, jnp.int32(0))
    pltpu.sync_copy(ob_v, out.at[pl.ds(off * 2, 2 * KP)])


_pred = pl.kernel(
    _pred_body,
    out_type=jax.ShapeDtypeStruct((2 * E,), jnp.float32),
    mesh=_MESH,
    compiler_params=_NLP,
    scratch_types=[
        pltpu.VMEM((NPT,), jnp.float32),
        pltpu.VMEM((KP,), jnp.int32),
        pltpu.VMEM((KP,), jnp.int32),
        pltpu.VMEM((2 * KP,), jnp.float32),
        pltpu.SemaphoreType.DMA,
    ],
)


def _pad_rows(x, rows):
    return jnp.pad(x, ((0, rows - x.shape[0]), (0, 0)))


def kernel(user_feat, item_feat, edge_index, W2, b2, W3, b3, W4, b4, W5, b5,
           Wp, bp):
    src = edge_index[0].astype(jnp.int32)
    dst = edge_index[1].astype(jnp.int32)
    z24 = jnp.zeros((RT, PW), jnp.int32)

    cm_packed = _cbuild(dst, src, z24)
    bytes4 = [((cm_packed >> (8 * k)) & 0xFF) for k in range(4)]
    cm = jnp.stack(bytes4, axis=-1).reshape(CR, CC).astype(jnp.bfloat16)

    ones_u = jnp.ones((N_U, 128), jnp.float32)
    xu_aug = _pad_rows(jnp.concatenate([user_feat, ones_u], axis=1), CC)
    xi_aug = _pad_rows(jnp.concatenate([item_feat, ones_u], axis=1), CR)
    si = _mm_n384(cm, xu_aug.astype(jnp.bfloat16))
    su = _mm_t384(cm, xi_aug.astype(jnp.bfloat16))
    sums_i = si[:N_I, :D]
    cnt_i = si[:N_I, D:D + 16]
    sums_u = su[:N_U, :D]
    cnt_u = su[:N_U, D:D + 16]

    u1 = _layer(user_feat, sums_u, cnt_u, W2[:D], W2[D:], b2.reshape(1, D))
    i1 = _layer(item_feat, sums_i, cnt_i, W3[:D], W3[D:], b3.reshape(1, D))

    si2 = _mm_n256(cm, _pad_rows(u1, CC).astype(jnp.bfloat16))
    su2 = _mm_t256(cm, _pad_rows(i1, CR).astype(jnp.bfloat16))

    wpa = jnp.pad(Wp[:D], ((0, 0), (0, 14)))
    wpb = jnp.pad(Wp[D:], ((0, 0), (0, 14)))
    bpu = jnp.pad(bp, (0, 14)).reshape(1, 16)
    zb = jnp.zeros((1, 16), jnp.float32)
    pu = _proj(u1, su2[:N_U], cnt_u, W4[:D], W4[D:], b4.reshape(1, D), wpa, bpu)
    pi = _proj(i1, si2[:N_I], cnt_i, W5[:D], W5[D:], b5.reshape(1, D), wpb, zb)

    pad = (0, CC - N_U)
    ptab = jnp.concatenate([
        jnp.pad(pu[:, 0], pad), jnp.pad(pu[:, 1], pad),
        jnp.pad(pi[:, 0], pad), jnp.pad(pi[:, 1], pad)])
    flat = _pred(ptab, src, dst)
    return flat.reshape(E, 2)


# speedup vs baseline: 5.8934x; 1.2833x over previous
import jax
import jax.numpy as jnp
import numpy as np
from jax import lax
from jax.experimental import pallas as pl
from jax.experimental.pallas import tpu as pltpu
from jax.experimental.pallas import tpu_sc as plsc

N_U = 5000
N_I = 5000
E = 160000
D = 256
NSUB = 16
CR = 5376
CC = 5120
PW = CC // 4
RT = 96
NSLOT = CR // RT
NPASS = 2
CHK = 1000
NCHK = E // CHK
GPC = CHK // 16

_Z = np.int32(0)
_MESH = plsc.VectorSubcoreMesh(core_axis_name="c", subcore_axis_name="s")
_NLP = pltpu.CompilerParams(needs_layout_passes=False)


def _cbuild_body(dsts, wcols, vals, z24, cout,
                 d0, w0b, v0, d1, w1b, v1, acc, sem0, sem1):
    c = lax.axis_index("c").astype(jnp.int32)
    s = lax.axis_index("s").astype(jnp.int32)
    w = s * jnp.int32(2) + c

    def load(g, db, wb, vb, sm):
        off = pl.multiple_of(g * jnp.int32(CHK), CHK)
        pltpu.async_copy(dsts.at[pl.ds(off, CHK)], db, sm)
        pltpu.async_copy(wcols.at[pl.ds(off, CHK)], wb, sm)
        pltpu.async_copy(vals.at[pl.ds(off, CHK)], vb, sm)

    def drain(db, wb, vb, sm):
        pltpu.make_async_copy(dsts.at[pl.ds(jnp.int32(0), CHK)], db, sm).wait()
        pltpu.make_async_copy(wcols.at[pl.ds(jnp.int32(0), CHK)], wb, sm).wait()
        pltpu.make_async_copy(vals.at[pl.ds(jnp.int32(0), CHK)], vb, sm).wait()

    def pass_body(p, carry):
        slot = p * jnp.int32(32) + w

        @pl.when(slot < jnp.int32(NSLOT))
        def _():
            row0 = slot * jnp.int32(RT)
            pltpu.sync_copy(z24, acc)
            row0v = jnp.zeros((16,), jnp.int32) + row0

            def process(db, wb, vb):
                def grp(j, cc2):
                    sl = pl.ds(j * jnp.int32(16), 16)
                    dv = db[sl]
                    lr = dv - row0v
                    mask = (lr >= 0) & (lr < jnp.int32(RT))
                    plsc.addupdate_scatter(acc, [lr, wb[sl]], vb[sl],
                                           mask=mask)
                    return cc2

                lax.fori_loop(jnp.int32(0), jnp.int32(GPC), grp, jnp.int32(0))

            load(jnp.int32(0), d0, w0b, v0, sem0)

            def two(gg, cc2):
                g0 = gg * jnp.int32(2)
                load(g0 + jnp.int32(1), d1, w1b, v1, sem1)
                drain(d0, w0b, v0, sem0)
                process(d0, w0b, v0)

                @pl.when(gg < jnp.int32(NCHK // 2 - 1))
                def _():
                    load(g0 + jnp.int32(2), d0, w0b, v0, sem0)

                drain(d1, w1b, v1, sem1)
                process(d1, w1b, v1)
                return cc2

            lax.fori_loop(jnp.int32(0), jnp.int32(NCHK // 2), two, jnp.int32(0))
            wout = pl.multiple_of(slot * jnp.int32(RT), 8)
            pltpu.sync_copy(acc, cout.at[pl.ds(wout, RT)])
        return carry

    lax.fori_loop(jnp.int32(0), jnp.int32(NPASS), pass_body, jnp.int32(0))


_cbuild = pl.kernel(
    _cbuild_body,
    out_type=jax.ShapeDtypeStruct((CR, PW), jnp.int32),
    mesh=_MESH,
    compiler_params=_NLP,
    scratch_types=[
        pltpu.VMEM((CHK,), jnp.int32),
        pltpu.VMEM((CHK,), jnp.int32),
        pltpu.VMEM((CHK,), jnp.int32),
        pltpu.VMEM((CHK,), jnp.int32),
        pltpu.VMEM((CHK,), jnp.int32),
        pltpu.VMEM((CHK,), jnp.int32),
        pltpu.VMEM((RT, PW), jnp.int32),
        pltpu.SemaphoreType.DMA,
        pltpu.SemaphoreType.DMA,
    ],
)


def _mm_n_kern(c_ref, x_ref, o_ref):
    cp = c_ref[:]
    cb = jnp.concatenate(
        [((cp >> (8 * k)) & 0xFF).astype(jnp.bfloat16) for k in range(4)],
        axis=1)
    o_ref[:] = jnp.dot(cb, x_ref[:], preferred_element_type=jnp.float32)


def _mm_t_kern(c_ref, x_ref, o_ref):
    plane = (pl.program_id(0) // (PW // _BMT)).astype(jnp.int32)
    cb = ((c_ref[:] >> (8 * plane)) & 0xFF).astype(jnp.bfloat16)
    o_ref[:] = lax.dot_general(cb, x_ref[:],
                               dimension_numbers=(((0,), (0,)), ((), ())),
                               preferred_element_type=jnp.float32)


_BMN = 384
_BMT = 256


def _make_mm_n(n):
    return pl.pallas_call(
        _mm_n_kern,
        grid=(CR // _BMN,),
        in_specs=[pl.BlockSpec((_BMN, PW), lambda i: (i, _Z)),
                  pl.BlockSpec((CC, n), lambda i: (_Z, _Z))],
        out_specs=pl.BlockSpec((_BMN, n), lambda i: (i, _Z)),
        out_shape=jax.ShapeDtypeStruct((CR, n), jnp.float32),
    )


def _make_mm_t(n):
    return pl.pallas_call(
        _mm_t_kern,
        grid=(CC // _BMT,),
        in_specs=[pl.BlockSpec((CR, _BMT), lambda i: (_Z, i % (PW // _BMT))),
                  pl.BlockSpec((CR, n), lambda i: (_Z, _Z))],
        out_specs=pl.BlockSpec((_BMT, n), lambda i: (i, _Z)),
        out_shape=jax.ShapeDtypeStruct((CC, n), jnp.float32),
    )


_mm_n384 = _make_mm_n(384)
_mm_t384 = _make_mm_t(384)
_mm_n256 = _make_mm_n(256)
_mm_t256 = _make_mm_t(256)


def _layer_kern(x_ref, s_ref, c_ref, wa_ref, wb_ref, b_ref, o_ref):
    inv = 1.0 / jnp.maximum(c_ref[:, 0:1], 1.0)
    h = s_ref[:] * inv
    y = jnp.dot(x_ref[:], wa_ref[:], preferred_element_type=jnp.float32)
    y = y + jnp.dot(h, wb_ref[:], preferred_element_type=jnp.float32)
    o_ref[:] = jnp.maximum(y + b_ref[:], 0.0)


def _proj_kern(x_ref, s_ref, c_ref, wa_ref, wb_ref, b_ref, wp_ref, pb_ref,
               o_ref):
    inv = 1.0 / jnp.maximum(c_ref[:, 0:1], 1.0)
    h = s_ref[:] * inv
    y = jnp.dot(x_ref[:], wa_ref[:], preferred_element_type=jnp.float32)
    y = y + jnp.dot(h, wb_ref[:], preferred_element_type=jnp.float32)
    y = y + b_ref[:]
    o_ref[:] = jnp.dot(y, wp_ref[:], preferred_element_type=jnp.float32) + pb_ref[:]


_BN = 1000


def _node_specs():
    return [
        pl.BlockSpec((_BN, D), lambda i: (i, _Z)),
        pl.BlockSpec((_BN, D), lambda i: (i, _Z)),
        pl.BlockSpec((_BN, 16), lambda i: (i, _Z)),
        pl.BlockSpec((D, D), lambda i: (_Z, _Z)),
        pl.BlockSpec((D, D), lambda i: (_Z, _Z)),
        pl.BlockSpec((1, D), lambda i: (_Z, _Z)),
    ]


_layer = pl.pallas_call(
    _layer_kern,
    grid=(N_U // _BN,),
    in_specs=_node_specs(),
    out_specs=pl.BlockSpec((_BN, D), lambda i: (i, _Z)),
    out_shape=jax.ShapeDtypeStruct((N_U, D), jnp.float32),
)

_proj = pl.pallas_call(
    _proj_kern,
    grid=(N_U // _BN,),
    in_specs=_node_specs() + [
        pl.BlockSpec((D, 16), lambda i: (_Z, _Z)),
        pl.BlockSpec((1, 16), lambda i: (_Z, _Z)),
    ],
    out_specs=pl.BlockSpec((_BN, 16), lambda i: (i, _Z)),
    out_shape=jax.ShapeDtypeStruct((N_U, 16), jnp.float32),
)


KP = 256
NCHP = E // KP
NPT = 4 * CC


def _pred_body(ptab, srcs, dsts, out, tab_v, si_v, di_v, ob_v, sem):
    c = lax.axis_index("c").astype(jnp.int32)
    s = lax.axis_index("s").astype(jnp.int32)
    w = s * jnp.int32(2) + c
    pltpu.sync_copy(ptab, tab_v)

    def step(i, carry):
        q = i * jnp.int32(2 * NSUB) + w

        @pl.when(q < jnp.int32(NCHP))
        def _():
            _pred_chunk(q, tab_v, si_v, di_v, ob_v, srcs, dsts, out, sem)
        return carry

    lax.fori_loop(jnp.int32(0), jnp.int32((NCHP + 2 * NSUB - 1) // (2 * NSUB)),
                  step, jnp.int32(0))


def _pred_chunk(q, tab_v, si_v, di_v, ob_v, srcs, dsts, out, sem):
    off = pl.multiple_of(q * jnp.int32(KP), KP)
    d1 = pltpu.async_copy(srcs.at[pl.ds(off, KP)], si_v, sem)
    d2 = pltpu.async_copy(dsts.at[pl.ds(off, KP)], di_v, sem)
    d1.wait()
    d2.wait()

    def inner(j, carry2):
        sv = si_v[pl.ds(j * jnp.int32(16), 16)]
        dv = di_v[pl.ds(j * jnp.int32(16), 16)]
        l0 = (plsc.load_gather(tab_v, [sv])
              + plsc.load_gather(tab_v, [dv + jnp.int32(2 * CC)]))
        l1 = (plsc.load_gather(tab_v, [sv + jnp.int32(CC)])
              + plsc.load_gather(tab_v, [dv + jnp.int32(3 * CC)]))
        m = jnp.maximum(l0, l1)
        e0 = jnp.exp(l0 - m)
        e1 = jnp.exp(l1 - m)
        t = e0 + e1
        jj = j * jnp.int32(32) + lax.iota(jnp.int32, 16) * jnp.int32(2)
        plsc.store_scatter(ob_v, [jj], e0 / t)
        plsc.store_scatter(ob_v, [jj + jnp.int32(1)], e1 / t)
        return carry2

    lax.fori_loop(jnp.int32(0), jnp.int32(KP // 16), inner, jnp.int32(0))
    pltpu.sync_copy(ob_v, out.at[pl.ds(off * 2, 2 * KP)])


_pred = pl.kernel(
    _pred_body,
    out_type=jax.ShapeDtypeStruct((2 * E,), jnp.float32),
    mesh=_MESH,
    compiler_params=_NLP,
    scratch_types=[
        pltpu.VMEM((NPT,), jnp.float32),
        pltpu.VMEM((KP,), jnp.int32),
        pltpu.VMEM((KP,), jnp.int32),
        pltpu.VMEM((2 * KP,), jnp.float32),
        pltpu.SemaphoreType.DMA,
    ],
)


def _pad_rows(x, rows):
    return jnp.pad(x, ((0, rows - x.shape[0]), (0, 0)))


def kernel(user_feat, item_feat, edge_index, W2, b2, W3, b3, W4, b4, W5, b5,
           Wp, bp):
    src = edge_index[0].astype(jnp.int32)
    dst = edge_index[1].astype(jnp.int32)
    z24 = jnp.zeros((RT, PW), jnp.int32)

    wcol = src % PW
    val = jnp.left_shift(jnp.int32(1), (src // PW) * 8)
    cm_packed = _cbuild(dst, wcol, val, z24)

    ones_u = jnp.ones((N_U, 128), jnp.float32)
    xu_aug = _pad_rows(jnp.concatenate([user_feat, ones_u], axis=1), CC)
    xi_aug = _pad_rows(jnp.concatenate([item_feat, ones_u], axis=1), CR)
    si = _mm_n384(cm_packed, xu_aug.astype(jnp.bfloat16))
    su = _mm_t384(cm_packed, xi_aug.astype(jnp.bfloat16))
    sums_i = si[:N_I, :D]
    cnt_i = si[:N_I, D:D + 16]
    sums_u = su[:N_U, :D]
    cnt_u = su[:N_U, D:D + 16]

    u1 = _layer(user_feat, sums_u, cnt_u, W2[:D], W2[D:], b2.reshape(1, D))
    i1 = _layer(item_feat, sums_i, cnt_i, W3[:D], W3[D:], b3.reshape(1, D))

    si2 = _mm_n256(cm_packed, _pad_rows(u1, CC).astype(jnp.bfloat16))
    su2 = _mm_t256(cm_packed, _pad_rows(i1, CR).astype(jnp.bfloat16))

    wpa = jnp.pad(Wp[:D], ((0, 0), (0, 14)))
    wpb = jnp.pad(Wp[D:], ((0, 0), (0, 14)))
    bpu = jnp.pad(bp, (0, 14)).reshape(1, 16)
    zb = jnp.zeros((1, 16), jnp.float32)
    pu = _proj(u1, su2[:N_U], cnt_u, W4[:D], W4[D:], b4.reshape(1, D), wpa, bpu)
    pi = _proj(i1, si2[:N_I], cnt_i, W5[:D], W5[D:], b5.reshape(1, D), wpb, zb)

    pad = (0, CC - N_U)
    ptab = jnp.concatenate([
        jnp.pad(pu[:, 0], pad), jnp.pad(pu[:, 1], pad),
        jnp.pad(pi[:, 0], pad), jnp.pad(pi[:, 1], pad)])
    flat = _pred(ptab, src, dst)
    return flat.reshape(E, 2)
